# baseline (speedup 1.0000x reference)
import jax
import jax.numpy as jnp
from jax.experimental import pallas as pl
from jax.experimental.pallas import tpu as pltpu

_K = 4
_R = 8
_C = 32
_P4, _P8 = 4, 8


def _iota2(shape, dim):
    return jax.lax.broadcasted_iota(jnp.int32, shape, dim)


def _col_to_row(col):
    n = col.shape[0]
    eye = (_iota2((n, n), 0) == _iota2((n, n), 1)).astype(jnp.float32)
    return jnp.sum(col * eye, axis=0, keepdims=True)


def _fused_kernel(x_ref, w6_ref, b6_ref, w7_ref, b7_ref, wcb_ref, bcb_ref,
                  wt_ref, bt_ref, w1_ref, b1_ref,
                  cls_ref, bbox_ref, quad_ref, acc,
                  *, hblk, wblk, nch):
    c = pl.program_id(1)
    x = x_ref[0, 0]
    h, w = x.shape
    inv = 1.0 / (nch * hblk * wblk)

    xs = jnp.sum(x.reshape(_P8, hblk, w), axis=1)
    s_right = (_iota2((w, _P8), 0) // wblk == _iota2((w, _P8), 1)
               ).astype(jnp.float32)
    part = jnp.dot(xs, s_right, preferred_element_type=jnp.float32) * inv

    @pl.when(c == 0)
    def _init():
        acc[...] = part

    @pl.when(c != 0)
    def _accum():
        acc[...] += part

    @pl.when(c == nch - 1)
    def _heads():
        pool8 = acc[...]

        e64 = (_iota2((64, _P8), 1) == _iota2((64, _P8), 0) // _P8
               ).astype(jnp.float32)
        m64 = (_iota2((64, _P8), 1) == _iota2((64, _P8), 0) % _P8
               ).astype(jnp.float32)
        p64 = jnp.sum(jnp.dot(e64, pool8, preferred_element_type=jnp.float32)
                      * m64, axis=1, keepdims=True)

        ki, ni = _iota2((16, 64), 0), _iota2((16, 64), 1)
        q4 = (((ni // _P8) // 2 == ki // _P4)
              & ((ni % _P8) // 2 == ki % _P4)).astype(jnp.float32) * 0.25
        pool4 = jnp.dot(q4, p64, preferred_element_type=jnp.float32)

        ki, mi = _iota2((16, 16 * _C), 0), _iota2((16, 16 * _C), 1)
        sel = jnp.where(mi // _C == ki,
                        1.0 + 0.01 * (mi % _C).astype(jnp.float32), 0.0)
        w6f = jnp.dot(sel, w6_ref[...],
                      preferred_element_type=jnp.float32)
        v = jnp.dot(_col_to_row(pool4), w6f,
                    preferred_element_type=jnp.float32)
        roi = 1.0 + 0.1 * _iota2((_R, 1), 0).astype(jnp.float32)
        hb = jnp.maximum(roi * v + b6_ref[...], 0.0)
        hb = jnp.maximum(jnp.dot(hb, w7_ref[...],
                                 preferred_element_type=jnp.float32)
                         + b7_ref[...], 0.0)
        box = jnp.dot(hb, wcb_ref[...],
                      preferred_element_type=jnp.float32) + bcb_ref[...]
        cls_ref[...] = box[:, :_K]
        bbox_ref[...] = box[:, _K:5 * _K]

        mi, ni = _iota2((_R * 64, 64), 0), _iota2((_R * 64, 64), 1)
        rmat = jnp.where(mi % 64 == ni,
                         1.0 + 0.1 * (mi // 64).astype(jnp.float32), 0.0)
        s = jnp.dot(rmat, p64, preferred_element_type=jnp.float32)
        ch_row = 1.0 + 0.01 * _iota2((1, _C), 1).astype(jnp.float32)
        u = jnp.dot(ch_row, wt_ref[...],
                    preferred_element_type=jnp.float32)
        hm = jnp.maximum(s * u + bt_ref[...], 0.0)
        q_full = jnp.dot(hm, w1_ref[...],
                         preferred_element_type=jnp.float32) + b1_ref[...]
        quad_ref[...] = q_full[:, :4 * _K]


def kernel(images, w6_pad, b6_pad, w7_pad, b7_pad, wcb_pad, bcb_pad,
           wt_cat, bt_cat, w1_bd_pad, b1_cat_pad):
    b, nch, h, w = images.shape
    hblk, wblk = h // _P8, w // _P8
    n_roi = b * _R
    mask_rows = _R * _P8 * _P8

    import functools
    body = functools.partial(_fused_kernel, hblk=hblk, wblk=wblk, nch=nch)

    bcast = lambda i, c: (0, 0)
    cls, bbox, quad = pl.pallas_call(
        body,
        out_shape=(jax.ShapeDtypeStruct((n_roi, _K), jnp.float32),
                   jax.ShapeDtypeStruct((n_roi, 4 * _K), jnp.float32),
                   jax.ShapeDtypeStruct((b * mask_rows, 4 * _K), jnp.float32)),
        grid_spec=pltpu.PrefetchScalarGridSpec(
            num_scalar_prefetch=0,
            grid=(b, nch),
            in_specs=[
                pl.BlockSpec((1, 1, h, w), lambda i, c: (i, c, 0, 0)),
                pl.BlockSpec(w6_pad.shape, bcast),
                pl.BlockSpec(b6_pad.shape, bcast),
                pl.BlockSpec(w7_pad.shape, bcast),
                pl.BlockSpec(b7_pad.shape, bcast),
                pl.BlockSpec(wcb_pad.shape, bcast),
                pl.BlockSpec(bcb_pad.shape, bcast),
                pl.BlockSpec(wt_cat.shape, bcast),
                pl.BlockSpec(bt_cat.shape, bcast),
                pl.BlockSpec(w1_bd_pad.shape, bcast),
                pl.BlockSpec(b1_cat_pad.shape, bcast),
            ],
            out_specs=(pl.BlockSpec((_R, _K), lambda i, c: (i, 0)),
                       pl.BlockSpec((_R, 4 * _K), lambda i, c: (i, 0)),
                       pl.BlockSpec((mask_rows, 4 * _K), lambda i, c: (i, 0))),
            scratch_shapes=[pltpu.VMEM((_P8, _P8), jnp.float32)],
        ),
        compiler_params=pltpu.CompilerParams(
            dimension_semantics=("parallel", "arbitrary")),
    )(images, w6_pad, b6_pad, w7_pad, b7_pad, wcb_pad, bcb_pad,
      wt_cat, bt_cat, w1_bd_pad, b1_cat_pad)

    out6 = quad.reshape(n_roi, _P8, _P8, 2, 2, _K)
    mask_logits = jnp.transpose(out6, (0, 5, 1, 3, 2, 4)).reshape(
        n_roi, _K, 2 * _P8, 2 * _P8)
    return cls, bbox, mask_logits


# speedup vs baseline: 1.0329x; 1.0329x over previous
import jax
import jax.numpy as jnp
from jax.experimental import pallas as pl
from jax.experimental.pallas import tpu as pltpu

_K = 4
_R = 8
_C = 32
_P4, _P8 = 4, 8


def _iota2(shape, dim):
    return jax.lax.broadcasted_iota(jnp.int32, shape, dim)


def _col_to_row(col):
    n = col.shape[0]
    eye = (_iota2((n, n), 0) == _iota2((n, n), 1)).astype(jnp.float32)
    return jnp.sum(col * eye, axis=0, keepdims=True)


def _fused_kernel(x_ref, w6_ref, b6_ref, w7_ref, b7_ref, wcb_ref, bcb_ref,
                  wt_ref, bt_ref, w1_ref, b1_ref,
                  cls_ref, bbox_ref, mask_ref, acc,
                  *, hblk, wblk, nch):
    c = pl.program_id(1)
    x = x_ref[0, 0]
    h, w = x.shape
    inv = 1.0 / (nch * hblk * wblk)

    s_left = (_iota2((_P8, h), 1) // hblk == _iota2((_P8, h), 0)
              ).astype(jnp.float32)
    s_right = (_iota2((w, _P8), 0) // wblk == _iota2((w, _P8), 1)
               ).astype(jnp.float32)
    rows = jnp.dot(s_left, x, preferred_element_type=jnp.float32)
    part = jnp.dot(rows, s_right, preferred_element_type=jnp.float32) * inv

    @pl.when(c == 0)
    def _init():
        acc[...] = part

    @pl.when(c != 0)
    def _accum():
        acc[...] += part

    @pl.when(c == nch - 1)
    def _heads():
        pool8 = acc[...]

        e64 = (_iota2((64, _P8), 1) == _iota2((64, _P8), 0) // _P8
               ).astype(jnp.float32)
        m64 = (_iota2((64, _P8), 1) == _iota2((64, _P8), 0) % _P8
               ).astype(jnp.float32)
        p64 = jnp.sum(jnp.dot(e64, pool8, preferred_element_type=jnp.float32)
                      * m64, axis=1, keepdims=True)

        ki, ni = _iota2((16, 64), 0), _iota2((16, 64), 1)
        q4 = (((ni // _P8) // 2 == ki // _P4)
              & ((ni % _P8) // 2 == ki % _P4)).astype(jnp.float32) * 0.25
        pool4 = jnp.dot(q4, p64, preferred_element_type=jnp.float32)

        ki, mi = _iota2((16, 16 * _C), 0), _iota2((16, 16 * _C), 1)
        sel = jnp.where(mi // _C == ki,
                        1.0 + 0.01 * (mi % _C).astype(jnp.float32), 0.0)
        w6f = jnp.dot(sel, w6_ref[...],
                      preferred_element_type=jnp.float32)
        v = jnp.dot(_col_to_row(pool4), w6f,
                    preferred_element_type=jnp.float32)
        roi = 1.0 + 0.1 * _iota2((_R, 1), 0).astype(jnp.float32)
        hb = jnp.maximum(roi * v + b6_ref[...], 0.0)
        hb = jnp.maximum(jnp.dot(hb, w7_ref[...],
                                 preferred_element_type=jnp.float32)
                         + b7_ref[...], 0.0)
        box = jnp.dot(hb, wcb_ref[...],
                      preferred_element_type=jnp.float32) + bcb_ref[...]
        cls_ref[...] = box[:, :_K]
        bbox_ref[...] = box[:, _K:5 * _K]

        mi, ni = _iota2((_R * 64, 64), 0), _iota2((_R * 64, 64), 1)
        rmat = jnp.where(mi % 64 == ni,
                         1.0 + 0.1 * (mi // 64).astype(jnp.float32), 0.0)
        s = jnp.dot(rmat, p64, preferred_element_type=jnp.float32)
        ch_row = 1.0 + 0.01 * _iota2((1, _C), 1).astype(jnp.float32)
        u = jnp.dot(ch_row, wt_ref[...],
                    preferred_element_type=jnp.float32)
        hm = jnp.maximum(s * u + bt_ref[...], 0.0)
        q_full = jnp.dot(hm, w1_ref[...],
                         preferred_element_type=jnp.float32) + b1_ref[...]
        t = q_full[:, :4 * _K].reshape(_R, _P8, _P8, 2, 2, _K)
        mask_ref[...] = jnp.transpose(t, (0, 5, 1, 3, 2, 4)).reshape(
            _R, _K, 2 * _P8, 2 * _P8)


def kernel(images, w6_pad, b6_pad, w7_pad, b7_pad, wcb_pad, bcb_pad,
           wt_cat, bt_cat, w1_bd_pad, b1_cat_pad):
    b, nch, h, w = images.shape
    hblk, wblk = h // _P8, w // _P8
    n_roi = b * _R
    mask_rows = _R * _P8 * _P8

    import functools
    body = functools.partial(_fused_kernel, hblk=hblk, wblk=wblk, nch=nch)

    bcast = lambda i, c: (0, 0)
    cls, bbox, mask_logits = pl.pallas_call(
        body,
        out_shape=(jax.ShapeDtypeStruct((n_roi, _K), jnp.float32),
                   jax.ShapeDtypeStruct((n_roi, 4 * _K), jnp.float32),
                   jax.ShapeDtypeStruct((n_roi, _K, 2 * _P8, 2 * _P8),
                                        jnp.float32)),
        grid_spec=pltpu.PrefetchScalarGridSpec(
            num_scalar_prefetch=0,
            grid=(b, nch),
            in_specs=[
                pl.BlockSpec((1, 1, h, w), lambda i, c: (i, c, 0, 0)),
                pl.BlockSpec(w6_pad.shape, bcast),
                pl.BlockSpec(b6_pad.shape, bcast),
                pl.BlockSpec(w7_pad.shape, bcast),
                pl.BlockSpec(b7_pad.shape, bcast),
                pl.BlockSpec(wcb_pad.shape, bcast),
                pl.BlockSpec(bcb_pad.shape, bcast),
                pl.BlockSpec(wt_cat.shape, bcast),
                pl.BlockSpec(bt_cat.shape, bcast),
                pl.BlockSpec(w1_bd_pad.shape, bcast),
                pl.BlockSpec(b1_cat_pad.shape, bcast),
            ],
            out_specs=(pl.BlockSpec((_R, _K), lambda i, c: (i, 0)),
                       pl.BlockSpec((_R, 4 * _K), lambda i, c: (i, 0)),
                       pl.BlockSpec((_R, _K, 2 * _P8, 2 * _P8),
                                    lambda i, c: (i, 0, 0, 0))),
            scratch_shapes=[pltpu.VMEM((_P8, _P8), jnp.float32)],
        ),
        compiler_params=pltpu.CompilerParams(
            dimension_semantics=("parallel", "arbitrary")),
    )(images, w6_pad, b6_pad, w7_pad, b7_pad, wcb_pad, bcb_pad,
      wt_cat, bt_cat, w1_bd_pad, b1_cat_pad)

    return cls, bbox, mask_logits


# speedup vs baseline: 1.0427x; 1.0095x over previous
import functools

import jax
import jax.numpy as jnp
from jax.experimental import pallas as pl
from jax.experimental.pallas import tpu as pltpu

_K = 4
_R = 8
_C = 32
_P4, _P8 = 4, 8
_HID = 4 * _C


def _iota2(shape, dim):
    return jax.lax.broadcasted_iota(jnp.int32, shape, dim)


def _col_to_row(col):
    n = col.shape[0]
    eye = (_iota2((n, n), 0) == _iota2((n, n), 1)).astype(jnp.float32)
    return jnp.sum(col * eye, axis=0, keepdims=True)


def _fdot(a, b):
    return jnp.dot(a, b, preferred_element_type=jnp.float32)


def _fused_kernel(x_ref, w6_ref, b6_ref, w7_ref, b7_ref, wcb_ref, bcb_ref,
                  wt_ref, bt_ref, w1_ref, b1_ref,
                  cls_ref, bbox_ref, mask_ref,
                  acc, w6f_s, k8_s, btt_s, wall_s, b1t_s, a64_s, ce_s, rm_s,
                  *, hblk, wblk, nch):
    c = pl.program_id(1)
    x = x_ref[0, 0]
    h, w = x.shape
    inv = 1.0 / (nch * hblk * wblk)

    s_left = (_iota2((_P8, h), 1) // hblk == _iota2((_P8, h), 0)
              ).astype(jnp.float32)
    s_right = (_iota2((w, _P8), 0) // wblk == _iota2((w, _P8), 1)
               ).astype(jnp.float32)
    part = _fdot(_fdot(s_left, x), s_right) * inv

    @pl.when(c == 0)
    def _init():
        acc[...] = part

    @pl.when(c != 0)
    def _accum():
        acc[...] += part

    @pl.when(c == 0)
    def _build():
        ki, mi = _iota2((16, 16 * _C), 0), _iota2((16, 16 * _C), 1)
        sel = jnp.where(mi // _C == ki,
                        1.0 + 0.01 * (mi % _C).astype(jnp.float32), 0.0)
        w6f_s[...] = _fdot(sel, w6_ref[...])

        ch_row = 1.0 + 0.01 * _iota2((1, _C), 1).astype(jnp.float32)
        u = _fdot(ch_row, wt_ref[...])
        tt = (_iota2((_HID, _R * _HID), 0) == _iota2((_HID, _R * _HID), 1)
              % _HID).astype(jnp.float32)
        k8_s[...] = jnp.where(
            _iota2((_P8, _R * _HID), 0) == _iota2((_P8, _R * _HID), 1) // _HID,
            _fdot(u, tt), 0.0)
        btt_s[...] = _fdot(bt_ref[...], tt)
        ttc = (_iota2((_R * _HID, _HID), 1) == _iota2((_R * _HID, _HID), 0)
               % _HID).astype(jnp.float32)
        w1s = w1_ref[...][:, :4 * _K]
        tl = (_iota2((4 * _K, _HID), 0) == _iota2((4 * _K, _HID), 1)
              % (4 * _K)).astype(jnp.float32)
        wall_s[...] = jnp.where(
            _iota2((_R * _HID, _HID), 0) // _HID
            == _iota2((_R * _HID, _HID), 1) // (4 * _K),
            _fdot(_fdot(ttc, w1s), tl), 0.0)
        b1t_s[...] = _fdot(b1_ref[...][:, :4 * _K], tl)
        a64_s[...] = jnp.where(
            _iota2((_R * _P8, _P8), 0) % _P8 == _iota2((_R * _P8, _P8), 1),
            1.0 + 0.1 * (_iota2((_R * _P8, _P8), 0) // _P8
                         ).astype(jnp.float32), 0.0)
        for idx in range(8):
            kc, di = idx // 2, idx % 2
            li, oj = _iota2((_HID, 2 * _P8), 0), _iota2((_HID, 2 * _P8), 1)
            ce = ((li % (4 * _K) == (2 * di + oj % 2) * _K + kc)
                  & (li // (4 * _K) == oj // 2)).astype(jnp.float32)
            ce_s[idx * _HID:(idx + 1) * _HID, :] = ce
            mo, mi2 = _iota2((512, _R * _P8), 0), _iota2((512, _R * _P8), 1)
            rm = ((mi2 // _P8 == mo // 64)
                  & (2 * (mi2 % _P8) + di == mo % 16)
                  & ((mo % 64) // 16 == kc)).astype(jnp.float32)
            rm_s[idx * 512:(idx + 1) * 512, :] = rm

    @pl.when(c == nch - 1)
    def _heads():
        pool8 = acc[...]

        e64 = (_iota2((64, _P8), 1) == _iota2((64, _P8), 0) // _P8
               ).astype(jnp.float32)
        m64 = (_iota2((64, _P8), 1) == _iota2((64, _P8), 0) % _P8
               ).astype(jnp.float32)
        p64 = jnp.sum(_fdot(e64, pool8) * m64, axis=1, keepdims=True)

        ki, ni = _iota2((16, 64), 0), _iota2((16, 64), 1)
        q4 = (((ni // _P8) // 2 == ki // _P4)
              & ((ni % _P8) // 2 == ki % _P4)).astype(jnp.float32) * 0.25
        pool4 = _fdot(q4, p64)

        v = _fdot(_col_to_row(pool4), w6f_s[...])
        roi = 1.0 + 0.1 * _iota2((_R, 1), 0).astype(jnp.float32)
        hb = jnp.maximum(roi * v + b6_ref[...], 0.0)
        hb = jnp.maximum(_fdot(hb, w7_ref[...]) + b7_ref[...], 0.0)
        box = _fdot(hb, wcb_ref[...]) + bcb_ref[...]
        cls_ref[...] = box[:, :_K]
        bbox_ref[...] = box[:, _K:5 * _K]

        s2 = _fdot(a64_s[...], pool8)
        hm2 = jnp.maximum(_fdot(s2, k8_s[...]) + btt_s[...], 0.0)
        sub = _fdot(hm2, wall_s[...]) + b1t_s[...]
        m2 = jnp.zeros((512, 2 * _P8), jnp.float32)
        for idx in range(8):
            p = _fdot(sub, ce_s[idx * _HID:(idx + 1) * _HID, :])
            m2 = m2 + _fdot(rm_s[idx * 512:(idx + 1) * 512, :], p)
        mask_ref[...] = m2.reshape(_R, _K, 2 * _P8, 2 * _P8)


def kernel(images, w6_pad, b6_pad, w7_pad, b7_pad, wcb_pad, bcb_pad,
           wt_cat, bt_cat, w1_bd_pad, b1_cat_pad):
    b, nch, h, w = images.shape
    hblk, wblk = h // _P8, w // _P8
    n_roi = b * _R

    body = functools.partial(_fused_kernel, hblk=hblk, wblk=wblk, nch=nch)

    bcast = lambda i, c: (0, 0)
    cls, bbox, mask_logits = pl.pallas_call(
        body,
        out_shape=(jax.ShapeDtypeStruct((n_roi, _K), jnp.float32),
                   jax.ShapeDtypeStruct((n_roi, 4 * _K), jnp.float32),
                   jax.ShapeDtypeStruct((n_roi, _K, 2 * _P8, 2 * _P8),
                                        jnp.float32)),
        grid_spec=pltpu.PrefetchScalarGridSpec(
            num_scalar_prefetch=0,
            grid=(b, nch),
            in_specs=[
                pl.BlockSpec((1, 1, h, w), lambda i, c: (i, c, 0, 0)),
                pl.BlockSpec(w6_pad.shape, bcast),
                pl.BlockSpec(b6_pad.shape, bcast),
                pl.BlockSpec(w7_pad.shape, bcast),
                pl.BlockSpec(b7_pad.shape, bcast),
                pl.BlockSpec(wcb_pad.shape, bcast),
                pl.BlockSpec(bcb_pad.shape, bcast),
                pl.BlockSpec(wt_cat.shape, bcast),
                pl.BlockSpec(bt_cat.shape, bcast),
                pl.BlockSpec(w1_bd_pad.shape, bcast),
                pl.BlockSpec(b1_cat_pad.shape, bcast),
            ],
            out_specs=(pl.BlockSpec((_R, _K), lambda i, c: (i, 0)),
                       pl.BlockSpec((_R, 4 * _K), lambda i, c: (i, 0)),
                       pl.BlockSpec((_R, _K, 2 * _P8, 2 * _P8),
                                    lambda i, c: (i, 0, 0, 0))),
            scratch_shapes=[
                pltpu.VMEM((_P8, _P8), jnp.float32),
                pltpu.VMEM((16, _HID), jnp.float32),
                pltpu.VMEM((_P8, _R * _HID), jnp.float32),
                pltpu.VMEM((1, _R * _HID), jnp.float32),
                pltpu.VMEM((_R * _HID, _HID), jnp.float32),
                pltpu.VMEM((1, _HID), jnp.float32),
                pltpu.VMEM((_R * _P8, _P8), jnp.float32),
                pltpu.VMEM((8 * _HID, 4 * _K), jnp.float32),
                pltpu.VMEM((8 * 512, _R * _P8), jnp.float32),
            ],
        ),
        compiler_params=pltpu.CompilerParams(
            dimension_semantics=("parallel", "arbitrary")),
    )(images, w6_pad, b6_pad, w7_pad, b7_pad, wcb_pad, bcb_pad,
      wt_cat, bt_cat, w1_bd_pad, b1_cat_pad)

    return cls, bbox, mask_logits
